# baseline (speedup 1.0000x reference)
import functools

import jax
import jax.numpy as jnp
from jax import lax
from jax.experimental import pallas as pl
from jax.experimental.pallas import tpu as pltpu
from jax.experimental.pallas import tpu_sc as plsc

LANES = 16
D_PAD = 128


@functools.lru_cache(maxsize=None)
def _build_transpose(n_emb: int, d_emb: int):
    info = plsc.get_sparse_core_info()
    nc, ns = info.num_cores, info.num_subcores
    nw = nc * ns
    n_full = n_emb // 128
    tail = n_emb - n_full * 128
    n_blk = n_full + (1 if tail else 0)
    n_pad = n_blk * 128
    blk_w = (n_full + nw - 1) // nw

    mesh = plsc.VectorSubcoreMesh(core_axis_name="c", subcore_axis_name="s")

    @functools.partial(
        pl.kernel,
        out_type=jax.ShapeDtypeStruct((n_pad, D_PAD), jnp.float32),
        mesh=mesh,
        compiler_params=pltpu.CompilerParams(needs_layout_passes=False),
        scratch_types=[
            pltpu.VMEM((d_emb, 128), jnp.float32),
            pltpu.VMEM((128, D_PAD), jnp.float32),
            pltpu.SemaphoreType.DMA,
        ],
    )
    def tpose(tt_hbm, tail_hbm, out_hbm, in_v, out_v, sem):
        wid = lax.axis_index("s") * nc + lax.axis_index("c")
        i16 = lax.iota(jnp.int32, 16)

        if tail:
            @pl.when(wid == nw - 1)
            def _tail():
                pltpu.sync_copy(tail_hbm, out_hbm.at[pl.ds(n_full * 128, 128), :])

        def block(g, carry):
            blk = wid + g * nw

            @pl.when(blk < n_full)
            def _go():
                pltpu.sync_copy(tt_hbm.at[:, pl.ds(blk * 128, 128)], in_v)

                def frow(f, c2):
                    for k in range(8):
                        v = in_v[f, pl.ds(k * 16, 16)]
                        plsc.store_scatter(
                            out_v, [i16 + (k * 16), jnp.full((16,), f, jnp.int32)], v
                        )
                    return c2

                lax.fori_loop(0, d_emb, frow, 0)
                pltpu.sync_copy(out_v, out_hbm.at[pl.ds(blk * 128, 128), :])

            return carry

        lax.fori_loop(0, blk_w, block, 0)

    return tpose


@functools.lru_cache(maxsize=None)
def _build_ebag(n_bags: int, per_bag: int, d_emb: int, n_pad: int):
    info = plsc.get_sparse_core_info()
    nc, ns = info.num_cores, info.num_subcores
    nw = nc * ns
    assert n_bags % nw == 0
    bags_w = n_bags // nw
    idx_w = bags_w * per_bag
    cb = 16
    assert bags_w % cb == 0
    ci = cb * per_bag
    nch = bags_w // cb
    nd = d_emb // LANES

    mesh = plsc.VectorSubcoreMesh(core_axis_name="c", subcore_axis_name="s")

    @functools.partial(
        pl.kernel,
        out_type=jax.ShapeDtypeStruct((n_bags, d_emb), jnp.float32),
        mesh=mesh,
        scratch_types=[
            pltpu.VMEM((idx_w,), jnp.int32),
            pltpu.VMEM((ci, D_PAD), jnp.float32),
            pltpu.VMEM((cb, d_emb), jnp.float32),
            pltpu.SemaphoreType.DMA,
        ],
    )
    def ebag(idx_hbm, table_hbm, out_hbm, idx_v, rows_v, ob_v, sem):
        wid = lax.axis_index("s") * nc + lax.axis_index("c")
        ibase = wid * idx_w
        obase = wid * bags_w
        pltpu.sync_copy(idx_hbm.at[pl.ds(ibase, idx_w)], idx_v)

        def chunk(g, carry):
            off = pl.multiple_of(g * ci, 8)
            pltpu.async_copy(
                table_hbm.at[idx_v.at[pl.ds(off, ci)]], rows_v, sem
            ).wait()

            def bag(b, c2):
                r0 = b * per_bag
                for dsub in range(nd):
                    sl = pl.ds(dsub * LANES, LANES)
                    acc = rows_v[r0, sl]
                    for j in range(1, per_bag):
                        acc = acc + rows_v[r0 + j, sl]
                    ob_v[b, sl] = acc
                return c2

            lax.fori_loop(0, cb, bag, 0)
            pltpu.sync_copy(ob_v, out_hbm.at[pl.ds(obase + g * cb, cb)])
            return carry

        lax.fori_loop(0, nch, chunk, 0)

    return ebag


def kernel(input, offset, table):
    n_bags = offset.shape[0]
    total = input.shape[0]
    per_bag = total // n_bags
    n_emb, d_emb = table.shape
    n_full = n_emb // 128
    n_tail = n_emb - n_full * 128
    n_pad = (n_full + (1 if n_tail else 0)) * 128
    tail_rows = jnp.zeros((128, D_PAD), jnp.float32)
    if n_tail:
        tail_rows = tail_rows.at[:n_tail, :d_emb].set(table[n_full * 128:, :])
    tpose = _build_transpose(n_emb, d_emb)
    ebag = _build_ebag(n_bags, per_bag, d_emb, n_pad)
    tp = tpose(table.T, tail_rows)
    return ebag(input.astype(jnp.int32), tp)


# speedup vs baseline: 1.2467x; 1.2467x over previous
import functools

import jax
import jax.numpy as jnp
from jax import lax
from jax.experimental import pallas as pl
from jax.experimental.pallas import tpu as pltpu
from jax.experimental.pallas import tpu_sc as plsc

LANES = 16
D_PAD = 128


@functools.lru_cache(maxsize=None)
def _build_transpose(n_emb: int, d_emb: int):
    info = plsc.get_sparse_core_info()
    nc, ns = info.num_cores, info.num_subcores
    nw = nc * ns
    n_full = n_emb // 128
    tail = n_emb - n_full * 128
    n_blk = n_full + (1 if tail else 0)
    n_pad = n_blk * 128
    blk_w = (n_full + nw - 1) // nw

    mesh = plsc.VectorSubcoreMesh(core_axis_name="c", subcore_axis_name="s")

    nb2 = (blk_w + 1) // 2

    @functools.partial(
        pl.kernel,
        out_type=jax.ShapeDtypeStruct((n_pad, D_PAD), jnp.float32),
        mesh=mesh,
        compiler_params=pltpu.CompilerParams(needs_layout_passes=False),
        scratch_types=[
            pltpu.VMEM((d_emb, 128), jnp.float32),
            pltpu.VMEM((d_emb, 128), jnp.float32),
            pltpu.VMEM((128, D_PAD), jnp.float32),
            pltpu.VMEM((128, D_PAD), jnp.float32),
            pltpu.SemaphoreType.DMA,
            pltpu.SemaphoreType.DMA,
            pltpu.SemaphoreType.DMA,
            pltpu.SemaphoreType.DMA,
        ],
    )
    def tpose(tt_hbm, tail_hbm, out_hbm, in0, in1, out0, out1, si0, si1, so0, so1):
        wid = lax.axis_index("s") * nc + lax.axis_index("c")
        i16 = lax.iota(jnp.int32, 16)

        if tail:
            @pl.when(wid == nw - 1)
            def _tail():
                pltpu.sync_copy(tail_hbm, out_hbm.at[pl.ds(n_full * 128, 128), :])

        def start_in(blk, inb, sem):
            @pl.when(blk < n_full)
            def _():
                pltpu.async_copy(tt_hbm.at[:, pl.ds(blk * 128, 128)], inb, sem)

        def wait_in(blk, inb, sem):
            @pl.when(blk < n_full)
            def _():
                pltpu.make_async_copy(tt_hbm.at[:, pl.ds(0, 128)], inb, sem).wait()

        def start_out(blk, outb, sem):
            @pl.when(blk < n_full)
            def _():
                pltpu.async_copy(outb, out_hbm.at[pl.ds(blk * 128, 128), :], sem)

        def wait_out(blk, outb, sem):
            @pl.when(blk < n_full)
            def _():
                pltpu.make_async_copy(
                    outb, out_hbm.at[pl.ds(0, 128), :], sem
                ).wait()

        def compute(blk, inb, outb):
            @pl.when(blk < n_full)
            def _():
                def frow(q, c2):
                    for j in range(4):
                        f = q * 4 + j
                        fcol = jnp.full((16,), f, jnp.int32)
                        for k in range(8):
                            v = inb[f, pl.ds(k * 16, 16)]
                            plsc.store_scatter(outb, [i16 + (k * 16), fcol], v)
                    return c2

                lax.fori_loop(0, d_emb // 4, frow, 0)

        start_in(wid, in0, si0)
        start_in(wid + nw, in1, si1)

        def body(h, carry):
            blk0 = wid + (2 * h) * nw
            blk1 = wid + (2 * h + 1) * nw
            wait_in(blk0, in0, si0)
            compute(blk0, in0, out0)
            start_out(blk0, out0, so0)
            start_in(blk0 + 2 * nw, in0, si0)
            wait_in(blk1, in1, si1)
            compute(blk1, in1, out1)
            start_out(blk1, out1, so1)
            start_in(blk1 + 2 * nw, in1, si1)
            wait_out(blk0, out0, so0)
            wait_out(blk1, out1, so1)
            return carry

        lax.fori_loop(0, nb2, body, 0)

    return tpose


@functools.lru_cache(maxsize=None)
def _build_ebag(n_bags: int, per_bag: int, d_emb: int, n_pad: int):
    info = plsc.get_sparse_core_info()
    nc, ns = info.num_cores, info.num_subcores
    nw = nc * ns
    assert n_bags % nw == 0
    bags_w = n_bags // nw
    idx_w = bags_w * per_bag
    cb = 16
    assert bags_w % cb == 0
    ci = cb * per_bag
    nch = bags_w // cb
    nd = d_emb // LANES

    mesh = plsc.VectorSubcoreMesh(core_axis_name="c", subcore_axis_name="s")

    @functools.partial(
        pl.kernel,
        out_type=jax.ShapeDtypeStruct((n_bags, d_emb), jnp.float32),
        mesh=mesh,
        scratch_types=[
            pltpu.VMEM((idx_w,), jnp.int32),
            pltpu.VMEM((ci, D_PAD), jnp.float32),
            pltpu.VMEM((cb, d_emb), jnp.float32),
            pltpu.SemaphoreType.DMA,
        ],
    )
    def ebag(idx_hbm, table_hbm, out_hbm, idx_v, rows_v, ob_v, sem):
        wid = lax.axis_index("s") * nc + lax.axis_index("c")
        ibase = wid * idx_w
        obase = wid * bags_w
        pltpu.sync_copy(idx_hbm.at[pl.ds(ibase, idx_w)], idx_v)

        def chunk(g, carry):
            off = pl.multiple_of(g * ci, 8)
            pltpu.async_copy(
                table_hbm.at[idx_v.at[pl.ds(off, ci)]], rows_v, sem
            ).wait()

            def bag(b, c2):
                r0 = b * per_bag
                for dsub in range(nd):
                    sl = pl.ds(dsub * LANES, LANES)
                    acc = rows_v[r0, sl]
                    for j in range(1, per_bag):
                        acc = acc + rows_v[r0 + j, sl]
                    ob_v[b, sl] = acc
                return c2

            lax.fori_loop(0, cb, bag, 0)
            pltpu.sync_copy(ob_v, out_hbm.at[pl.ds(obase + g * cb, cb)])
            return carry

        lax.fori_loop(0, nch, chunk, 0)

    return ebag


def kernel(input, offset, table):
    n_bags = offset.shape[0]
    total = input.shape[0]
    per_bag = total // n_bags
    n_emb, d_emb = table.shape
    n_full = n_emb // 128
    n_tail = n_emb - n_full * 128
    n_pad = (n_full + (1 if n_tail else 0)) * 128
    tail_rows = jnp.zeros((128, D_PAD), jnp.float32)
    if n_tail:
        tail_rows = tail_rows.at[:n_tail, :d_emb].set(table[n_full * 128:, :])
    tpose = _build_transpose(n_emb, d_emb)
    ebag = _build_ebag(n_bags, per_bag, d_emb, n_pad)
    tp = tpose(table.T, tail_rows)
    return ebag(input.astype(jnp.int32), tp)


# speedup vs baseline: 2.6241x; 2.1048x over previous
import functools

import jax
import jax.numpy as jnp
from jax import lax
from jax.experimental import pallas as pl
from jax.experimental.pallas import tpu as pltpu
from jax.experimental.pallas import tpu_sc as plsc

LANES = 16
D_PAD = 128


@functools.lru_cache(maxsize=None)
def _build_transpose(n_emb: int, d_emb: int):
    info = plsc.get_sparse_core_info()
    nc, ns = info.num_cores, info.num_subcores
    nw = nc * ns
    n_full = n_emb // 128
    tail = n_emb - n_full * 128
    n_blk = n_full + (1 if tail else 0)
    n_pad = n_blk * 128
    blk_w = (n_full + nw - 1) // nw

    mesh = plsc.VectorSubcoreMesh(core_axis_name="c", subcore_axis_name="s")

    nb2 = (blk_w + 1) // 2

    @functools.partial(
        pl.kernel,
        out_type=jax.ShapeDtypeStruct((n_pad, D_PAD), jnp.float32),
        mesh=mesh,
        compiler_params=pltpu.CompilerParams(needs_layout_passes=False),
        scratch_types=[
            pltpu.VMEM((d_emb, 128), jnp.float32),
            pltpu.VMEM((d_emb, 128), jnp.float32),
            pltpu.VMEM((128, D_PAD), jnp.float32),
            pltpu.VMEM((128, D_PAD), jnp.float32),
            pltpu.SemaphoreType.DMA,
            pltpu.SemaphoreType.DMA,
            pltpu.SemaphoreType.DMA,
            pltpu.SemaphoreType.DMA,
        ],
    )
    def tpose(tt_hbm, tail_hbm, out_hbm, in0, in1, out0, out1, si0, si1, so0, so1):
        wid = lax.axis_index("s") * nc + lax.axis_index("c")
        i16 = lax.iota(jnp.int32, 16)

        if tail:
            @pl.when(wid == nw - 1)
            def _tail():
                pltpu.sync_copy(tail_hbm, out_hbm.at[pl.ds(n_full * 128, 128), :])

        def start_in(blk, inb, sem):
            @pl.when(blk < n_full)
            def _():
                pltpu.async_copy(tt_hbm.at[:, pl.ds(blk * 128, 128)], inb, sem)

        def wait_in(blk, inb, sem):
            @pl.when(blk < n_full)
            def _():
                pltpu.make_async_copy(tt_hbm.at[:, pl.ds(0, 128)], inb, sem).wait()

        def start_out(blk, outb, sem):
            @pl.when(blk < n_full)
            def _():
                pltpu.async_copy(outb, out_hbm.at[pl.ds(blk * 128, 128), :], sem)

        def wait_out(blk, outb, sem):
            @pl.when(blk < n_full)
            def _():
                pltpu.make_async_copy(
                    outb, out_hbm.at[pl.ds(0, 128), :], sem
                ).wait()

        rows = [i16 + (k * 16) for k in range(8)]

        def compute(blk, inb, outb):
            @pl.when(blk < n_full)
            def _():
                @functools.partial(
                    plsc.parallel_loop, 0, d_emb, unroll=4,
                    carry=jnp.zeros((16,), jnp.int32),
                )
                def _body(f, fcol):
                    for k in range(8):
                        v = inb[f, pl.ds(k * 16, 16)]
                        plsc.store_scatter(outb, [rows[k], fcol], v)
                    return fcol + 1

        start_in(wid, in0, si0)
        start_in(wid + nw, in1, si1)

        def body(h, carry):
            blk0 = wid + (2 * h) * nw
            blk1 = wid + (2 * h + 1) * nw
            wait_in(blk0, in0, si0)
            compute(blk0, in0, out0)
            start_out(blk0, out0, so0)
            start_in(blk0 + 2 * nw, in0, si0)
            wait_in(blk1, in1, si1)
            compute(blk1, in1, out1)
            start_out(blk1, out1, so1)
            start_in(blk1 + 2 * nw, in1, si1)
            wait_out(blk0, out0, so0)
            wait_out(blk1, out1, so1)
            return carry

        lax.fori_loop(0, nb2, body, 0)

    return tpose


@functools.lru_cache(maxsize=None)
def _build_ebag(n_bags: int, per_bag: int, d_emb: int, n_pad: int):
    info = plsc.get_sparse_core_info()
    nc, ns = info.num_cores, info.num_subcores
    nw = nc * ns
    assert n_bags % nw == 0
    bags_w = n_bags // nw
    idx_w = bags_w * per_bag
    cb = 16
    assert bags_w % cb == 0
    ci = cb * per_bag
    nch = bags_w // cb
    nd = d_emb // LANES

    mesh = plsc.VectorSubcoreMesh(core_axis_name="c", subcore_axis_name="s")

    @functools.partial(
        pl.kernel,
        out_type=jax.ShapeDtypeStruct((n_bags, d_emb), jnp.float32),
        mesh=mesh,
        scratch_types=[
            pltpu.VMEM((idx_w,), jnp.int32),
            pltpu.VMEM((ci, D_PAD), jnp.float32),
            pltpu.VMEM((ci, D_PAD), jnp.float32),
            pltpu.VMEM((cb, d_emb), jnp.float32),
            pltpu.VMEM((cb, d_emb), jnp.float32),
            pltpu.SemaphoreType.DMA,
            pltpu.SemaphoreType.DMA,
            pltpu.SemaphoreType.DMA,
            pltpu.SemaphoreType.DMA,
        ],
    )
    def ebag(idx_hbm, table_hbm, out_hbm, idx_v, rows0, rows1, ob0, ob1,
             sg0, sg1, so0, so1):
        wid = lax.axis_index("s") * nc + lax.axis_index("c")
        ibase = wid * idx_w
        obase = wid * bags_w
        pltpu.sync_copy(idx_hbm.at[pl.ds(ibase, idx_w)], idx_v)

        def chunk(g, carry):
            off = pl.multiple_of(g * ci, 8)
            pltpu.async_copy(
                table_hbm.at[idx_v.at[pl.ds(off, ci)]], rows0, sg0
            ).wait()

            def bag(b, c2):
                r0 = b * per_bag
                for dsub in range(nd):
                    sl = pl.ds(dsub * LANES, LANES)
                    acc = rows0[r0, sl]
                    for j in range(1, per_bag):
                        acc = acc + rows0[r0 + j, sl]
                    ob0[b, sl] = acc
                return c2

            lax.fori_loop(0, cb, bag, 0)
            pltpu.sync_copy(ob0, out_hbm.at[pl.ds(obase + g * cb, cb)])
            return carry

        lax.fori_loop(0, nch, chunk, 0)

    return ebag


def kernel(input, offset, table):
    n_bags = offset.shape[0]
    total = input.shape[0]
    per_bag = total // n_bags
    n_emb, d_emb = table.shape
    tp = jnp.pad(table, ((0, 0), (0, D_PAD - d_emb)))
    ebag = _build_ebag(n_bags, per_bag, d_emb, n_emb)
    return ebag(input.astype(jnp.int32), tp)
